# baseline (speedup 1.0000x reference)
import functools

import jax
import jax.numpy as jnp
from jax import lax
from jax.experimental import pallas as pl
from jax.experimental.pallas import tpu as pltpu
from jax.experimental.pallas import tpu_sc as plsc

N = 10000
E = 320000
D = 128
G = 128

NC = 2
NS = 16
NW = NC * NS

NP = 10240
RPS = NP // NS

C = 128
EPAD = 327680
EW = EPAD // NW
NCH = EW // C
NDR = RPS // C

BN = 2000

_F32 = jnp.float32


def _sc_segsum(feat, src3, dst3, zeros):
    mesh = plsc.VectorSubcoreMesh(core_axis_name="c", subcore_axis_name="s")

    @functools.partial(
        pl.kernel,
        out_type=jax.ShapeDtypeStruct((NC * NP, D), _F32),
        mesh=mesh,
        scratch_types=[
            pltpu.VMEM((NCH, C), jnp.int32),
            pltpu.VMEM((NCH, C), jnp.int32),
            pltpu.VMEM((C, D), _F32),
            pltpu.VMEM_SHARED((NP, D), _F32),
            pltpu.SemaphoreType.DMA,
        ],
    )
    def k(feat_h, src_h, dst_h, zero_h, out_h, isrc, idst, rows, shared, sem):
        c = lax.axis_index("c")
        s = lax.axis_index("s")
        wid = s * NC + c
        pltpu.sync_copy(zero_h, rows)
        for t in range(NDR):
            pltpu.sync_copy(rows, shared.at[pl.ds(s * RPS + t * C, C)])
        pltpu.sync_copy(src_h.at[wid], isrc)
        pltpu.sync_copy(dst_h.at[wid], idst)
        plsc.subcore_barrier()

        def body(j, carry):
            pltpu.async_copy(feat_h.at[isrc.at[j]], rows, sem).wait()
            pltpu.sync_copy(rows, shared.at[idst.at[j]], add=True)
            return carry

        lax.fori_loop(0, NCH, body, 0)
        plsc.subcore_barrier()
        for t in range(NDR):
            pltpu.sync_copy(shared.at[pl.ds(s * RPS + t * C, C)], rows)
            pltpu.sync_copy(rows, out_h.at[pl.ds(c * NP + s * RPS + t * C, C)])

    return k(feat, src3, dst3, zeros)


def _sc_segcnt(dst3, zeros, ones):
    mesh = plsc.VectorSubcoreMesh(core_axis_name="c", subcore_axis_name="s")

    @functools.partial(
        pl.kernel,
        out_type=jax.ShapeDtypeStruct((NC * NP, D), _F32),
        mesh=mesh,
        scratch_types=[
            pltpu.VMEM((NCH, C), jnp.int32),
            pltpu.VMEM((C, D), _F32),
            pltpu.VMEM_SHARED((NP, D), _F32),
        ],
    )
    def k(dst_h, zero_h, ones_h, out_h, idst, rows, shared):
        c = lax.axis_index("c")
        s = lax.axis_index("s")
        wid = s * NC + c
        pltpu.sync_copy(zero_h, rows)
        for t in range(NDR):
            pltpu.sync_copy(rows, shared.at[pl.ds(s * RPS + t * C, C)])
        pltpu.sync_copy(ones_h, rows)
        pltpu.sync_copy(dst_h.at[wid], idst)
        plsc.subcore_barrier()

        def body(j, carry):
            pltpu.sync_copy(rows, shared.at[idst.at[j]], add=True)
            return carry

        lax.fori_loop(0, NCH, body, 0)
        plsc.subcore_barrier()
        for t in range(NDR):
            pltpu.sync_copy(shared.at[pl.ds(s * RPS + t * C, C)], rows)
            pltpu.sync_copy(rows, out_h.at[pl.ds(c * NP + s * RPS + t * C, C)])

    return k(dst3, zeros, ones)


def _sc_segsum_wide(feat_a, feat_b, src4, dst4, zeros):
    mesh = plsc.VectorSubcoreMesh(core_axis_name="c", subcore_axis_name="s")

    @functools.partial(
        pl.kernel,
        out_type=jax.ShapeDtypeStruct((NC * NP, D), _F32),
        mesh=mesh,
        scratch_types=[
            pltpu.VMEM((NCH, C), jnp.int32),
            pltpu.VMEM((NCH, C), jnp.int32),
            pltpu.VMEM((C, D), _F32),
            pltpu.VMEM_SHARED((NP, D), _F32),
            pltpu.SemaphoreType.DMA,
        ],
    )
    def k(fa_h, fb_h, src_h, dst_h, zero_h, out_h, isrc, idst, rows, shared, sem):
        c = lax.axis_index("c")
        s = lax.axis_index("s")
        pltpu.sync_copy(zero_h, rows)
        for t in range(NDR):
            pltpu.sync_copy(rows, shared.at[pl.ds(s * RPS + t * C, C)])
        plsc.subcore_barrier()

        def run(feat_h):
            for stage in range(2):
                pltpu.sync_copy(src_h.at[s, stage], isrc)
                pltpu.sync_copy(dst_h.at[s, stage], idst)

                def body(j, carry):
                    pltpu.async_copy(feat_h.at[isrc.at[j]], rows, sem).wait()
                    pltpu.sync_copy(rows, shared.at[idst.at[j]], add=True)
                    return carry

                lax.fori_loop(0, NCH, body, 0)

        @pl.when(c == 0)
        def _():
            run(fa_h)

        @pl.when(c == 1)
        def _():
            run(fb_h)

        plsc.subcore_barrier()
        for t in range(NDR):
            pltpu.sync_copy(shared.at[pl.ds(s * RPS + t * C, C)], rows)
            pltpu.sync_copy(rows, out_h.at[pl.ds(c * NP + s * RPS + t * C, C)])

    return k(feat_a, feat_b, src4, dst4, zeros)


def _dgT(a, b):
    return lax.dot_general(a, b, (((1,), (1,)), ((), ())),
                           preferred_element_type=_F32)


def _tc_layer(h, m0, m1, c0, c1, Wl, bl, Wr):
    din = h.shape[1]
    dout = Wl.shape[0]

    def body(h_r, m0_r, m1_r, c0_r, c1_r, wl_r, bl_r, wr_r, ho_r):
        cnt = jnp.maximum(c0_r[:, 0:1] + c1_r[:, 0:1], 1.0)
        agg = (m0_r[...] + m1_r[...]) / cnt
        hv = _dgT(h_r[...], wl_r[...]) + bl_r[...] + _dgT(agg, wr_r[...])
        ho_r[...] = jnp.maximum(hv, 0.0)

    in_specs = [
        pl.BlockSpec((BN, din), lambda i: (i, 0)),
        pl.BlockSpec((BN, 128), lambda i: (i, 0)),
        pl.BlockSpec((BN, 128), lambda i: (i, 0)),
        pl.BlockSpec((BN, 128), lambda i: (i, 0)),
        pl.BlockSpec((BN, 128), lambda i: (i, 0)),
        pl.BlockSpec((dout, din), lambda i: (0, 0)),
        pl.BlockSpec((1, dout), lambda i: (0, 0)),
        pl.BlockSpec((dout, 128), lambda i: (0, 0)),
    ]
    return pl.pallas_call(
        body,
        grid=(N // BN,),
        in_specs=in_specs,
        out_specs=pl.BlockSpec((BN, dout), lambda i: (i, 0)),
        out_shape=jax.ShapeDtypeStruct((N, dout), _F32),
    )(h, m0, m1, c0, c1, Wl, bl.reshape(1, -1), Wr)


def _tc_layer_split(h, ma, mb, c0, c1, Wl, bl, Wra, Wrb):
    din = h.shape[1]
    dout = Wl.shape[0]

    def body(h_r, ma_r, mb_r, c0_r, c1_r, wl_r, bl_r, wra_r, wrb_r, ho_r):
        cnt = jnp.maximum(c0_r[:, 0:1] + c1_r[:, 0:1], 1.0)
        agg = _dgT(ma_r[...] / cnt, wra_r[...]) + _dgT(mb_r[...] / cnt, wrb_r[...])
        hv = _dgT(h_r[...], wl_r[...]) + bl_r[...] + agg
        ho_r[...] = jnp.maximum(hv, 0.0)

    in_specs = [
        pl.BlockSpec((BN, din), lambda i: (i, 0)),
        pl.BlockSpec((BN, 128), lambda i: (i, 0)),
        pl.BlockSpec((BN, 128), lambda i: (i, 0)),
        pl.BlockSpec((BN, 128), lambda i: (i, 0)),
        pl.BlockSpec((BN, 128), lambda i: (i, 0)),
        pl.BlockSpec((dout, din), lambda i: (0, 0)),
        pl.BlockSpec((1, dout), lambda i: (0, 0)),
        pl.BlockSpec((dout, 128), lambda i: (0, 0)),
        pl.BlockSpec((dout, 128), lambda i: (0, 0)),
    ]
    return pl.pallas_call(
        body,
        grid=(N // BN,),
        in_specs=in_specs,
        out_specs=pl.BlockSpec((BN, dout), lambda i: (i, 0)),
        out_shape=jax.ShapeDtypeStruct((N, dout), _F32),
    )(h, ma, mb, c0, c1, Wl, bl.reshape(1, -1), Wra, Wrb)


def _tc_pool_mlp(h4, batch2, l1W, l1b, l2W, l2b, l3W, l3b, l4W, l4b):
    nblk = N // BN

    def body(h_r, b_r, w1, b1, w2, b2, w3, b3, w4, b4, out_r, acc, cacc):
        i = pl.program_id(0)

        @pl.when(i == 0)
        def _():
            acc[...] = jnp.zeros((G, D), _F32)
            cacc[...] = jnp.zeros((G, D), _F32)

        mask = (b_r[...] == lax.broadcasted_iota(jnp.int32, (BN, G), 1)).astype(_F32)
        acc[...] += lax.dot_general(mask, h_r[...], (((0,), (0,)), ((), ())),
                                    preferred_element_type=_F32,
                                    precision=lax.Precision.HIGHEST)
        cacc[...] += lax.dot_general(mask, jnp.ones((BN, D), _F32),
                                     (((0,), (0,)), ((), ())),
                                     preferred_element_type=_F32,
                                     precision=lax.Precision.HIGHEST)

        @pl.when(i == nblk - 1)
        def _():
            g = acc[...] / jnp.maximum(cacc[...], 1.0)
            g = jnp.maximum(_dgT(g, w1[...]) + b1[...], 0.0)
            g = jnp.maximum(_dgT(g, w2[...]) + b2[...], 0.0)
            g = jnp.maximum(_dgT(g, w3[...]) + b3[...], 0.0)
            o = lax.dot_general(w4[...], g, (((1,), (1,)), ((), ())),
                                preferred_element_type=_F32)
            out_r[...] = o + b4[...]

    in_specs = [
        pl.BlockSpec((BN, D), lambda i: (i, 0)),
        pl.BlockSpec((BN, 1), lambda i: (i, 0)),
        pl.BlockSpec((128, 128), lambda i: (0, 0)),
        pl.BlockSpec((1, 128), lambda i: (0, 0)),
        pl.BlockSpec((64, 128), lambda i: (0, 0)),
        pl.BlockSpec((1, 64), lambda i: (0, 0)),
        pl.BlockSpec((64, 64), lambda i: (0, 0)),
        pl.BlockSpec((1, 64), lambda i: (0, 0)),
        pl.BlockSpec((1, 64), lambda i: (0, 0)),
        pl.BlockSpec((1, 1), lambda i: (0, 0)),
    ]
    out = pl.pallas_call(
        body,
        grid=(nblk,),
        in_specs=in_specs,
        out_specs=pl.BlockSpec((1, G), lambda i: (0, 0)),
        out_shape=jax.ShapeDtypeStruct((1, G), _F32),
        scratch_shapes=[pltpu.VMEM((G, D), _F32), pltpu.VMEM((G, D), _F32)],
    )(h4, batch2,
      l1W, l1b.reshape(1, -1), l2W, l2b.reshape(1, -1),
      l3W, l3b.reshape(1, -1), l4W, l4b.reshape(1, -1))
    return out


def kernel(x, edge_index, batch,
           conv1_Wl, conv1_bl, conv1_Wr,
           conv2_Wl, conv2_bl, conv2_Wr,
           conv3_Wl, conv3_bl, conv3_Wr,
           conv4_Wl, conv4_bl, conv4_Wr,
           lin1_W, lin1_b, lin2_W, lin2_b,
           lin3_W, lin3_b, lin4_W, lin4_b):
    pad = EPAD - E
    src3 = jnp.concatenate(
        [edge_index[0], jnp.zeros((pad,), jnp.int32)]).reshape(NW, NCH, C)
    dst3 = jnp.concatenate(
        [edge_index[1], jnp.full((pad,), N, jnp.int32)]).reshape(NW, NCH, C)
    zeros128 = jnp.zeros((C, D), _F32)
    ones128 = jnp.ones((C, D), _F32)

    cnt = _sc_segcnt(dst3, zeros128, ones128)
    c0 = cnt[:N]
    c1 = cnt[NP:NP + N]

    def seg(feat):
        m = _sc_segsum(feat, src3, dst3, zeros128)
        return m[:N], m[NP:NP + N]

    src4 = src3.reshape(NS, 2, NCH, C)
    dst4 = dst3.reshape(NS, 2, NCH, C)

    m0, m1 = seg(x)
    h1 = _tc_layer(x, m0, m1, c0, c1, conv1_Wl, conv1_bl, conv1_Wr)
    mw = _sc_segsum_wide(h1[:, :128], h1[:, 128:], src4, dst4, zeros128)
    h2 = _tc_layer_split(h1, mw[:N], mw[NP:NP + N], c0, c1, conv2_Wl,
                         conv2_bl, conv2_Wr[:, :128], conv2_Wr[:, 128:])
    m0, m1 = seg(h2)
    h3 = _tc_layer(h2, m0, m1, c0, c1, conv3_Wl, conv3_bl, conv3_Wr)
    m0, m1 = seg(h3)
    h4 = _tc_layer(h3, m0, m1, c0, c1, conv4_Wl, conv4_bl, conv4_Wr)

    out = _tc_pool_mlp(h4, batch.reshape(N, 1),
                       lin1_W, lin1_b, lin2_W, lin2_b,
                       lin3_W, lin3_b, lin4_W, lin4_b)
    return out.reshape(G)


# speedup vs baseline: 1.1496x; 1.1496x over previous
import functools

import jax
import jax.numpy as jnp
from jax import lax
from jax.experimental import pallas as pl
from jax.experimental.pallas import tpu as pltpu
from jax.experimental.pallas import tpu_sc as plsc

N = 10000
E = 320000
D = 128
G = 128

NC = 2
NS = 16
NW = NC * NS

NP = 10240
RPS = NP // NS

C = 128
EPAD = 327680
EW = EPAD // NW
NCH = EW // C
NDR = RPS // C

BN = 2000

_F32 = jnp.float32


def _sc_segsum(feat, src3, dst3, zeros):
    mesh = plsc.VectorSubcoreMesh(core_axis_name="c", subcore_axis_name="s")

    @functools.partial(
        pl.kernel,
        out_type=jax.ShapeDtypeStruct((NC * NP, D), _F32),
        mesh=mesh,
        scratch_types=[
            pltpu.VMEM((C,), jnp.int32),
            pltpu.VMEM((C,), jnp.int32),
            pltpu.VMEM((C,), jnp.int32),
            pltpu.VMEM((C,), jnp.int32),
            pltpu.VMEM((C, D), _F32),
            pltpu.VMEM((C, D), _F32),
            pltpu.VMEM_SHARED((NP, D), _F32),
            pltpu.SemaphoreType.DMA,
            pltpu.SemaphoreType.DMA,
        ],
    )
    def k(feat_h, src_h, dst_h, zero_h, out_h,
          isa, ida, isb, idb, rows_a, rows_b, shared, sem_a, sem_b):
        c = lax.axis_index("c")
        s = lax.axis_index("s")
        wid = s * NC + c
        pltpu.sync_copy(zero_h, rows_a)
        for t in range(NDR):
            pltpu.sync_copy(rows_a, shared.at[pl.ds(s * RPS + t * C, C)])
        plsc.subcore_barrier()

        pltpu.sync_copy(src_h.at[wid, 0], isa)
        pltpu.sync_copy(dst_h.at[wid, 0], ida)
        pltpu.async_copy(feat_h.at[isa], rows_a, sem_a)

        def body(i, carry):
            j0 = 2 * i
            pltpu.sync_copy(src_h.at[wid, j0 + 1], isb)
            pltpu.sync_copy(dst_h.at[wid, j0 + 1], idb)
            pltpu.async_copy(feat_h.at[isb], rows_b, sem_b)
            pltpu.make_async_copy(feat_h.at[isa], rows_a, sem_a).wait()
            pltpu.sync_copy(rows_a, shared.at[ida], add=True)

            @pl.when(i < NCH // 2 - 1)
            def _():
                pltpu.sync_copy(src_h.at[wid, j0 + 2], isa)
                pltpu.sync_copy(dst_h.at[wid, j0 + 2], ida)
                pltpu.async_copy(feat_h.at[isa], rows_a, sem_a)

            pltpu.make_async_copy(feat_h.at[isb], rows_b, sem_b).wait()
            pltpu.sync_copy(rows_b, shared.at[idb], add=True)
            return carry

        lax.fori_loop(0, NCH // 2, body, 0)
        plsc.subcore_barrier()
        for t in range(NDR):
            pltpu.sync_copy(shared.at[pl.ds(s * RPS + t * C, C)], rows_a)
            pltpu.sync_copy(rows_a, out_h.at[pl.ds(c * NP + s * RPS + t * C, C)])

    return k(feat, src3, dst3, zeros)


def _sc_segcnt(dst3, zeros, ones):
    mesh = plsc.VectorSubcoreMesh(core_axis_name="c", subcore_axis_name="s")

    @functools.partial(
        pl.kernel,
        out_type=jax.ShapeDtypeStruct((NC * NP, D), _F32),
        mesh=mesh,
        scratch_types=[
            pltpu.VMEM((NCH, C), jnp.int32),
            pltpu.VMEM((C, D), _F32),
            pltpu.VMEM_SHARED((NP, D), _F32),
        ],
    )
    def k(dst_h, zero_h, ones_h, out_h, idst, rows, shared):
        c = lax.axis_index("c")
        s = lax.axis_index("s")
        wid = s * NC + c
        pltpu.sync_copy(zero_h, rows)
        for t in range(NDR):
            pltpu.sync_copy(rows, shared.at[pl.ds(s * RPS + t * C, C)])
        pltpu.sync_copy(ones_h, rows)
        pltpu.sync_copy(dst_h.at[wid], idst)
        plsc.subcore_barrier()

        def body(j, carry):
            pltpu.sync_copy(rows, shared.at[idst.at[j]], add=True)
            return carry

        lax.fori_loop(0, NCH, body, 0)
        plsc.subcore_barrier()
        for t in range(NDR):
            pltpu.sync_copy(shared.at[pl.ds(s * RPS + t * C, C)], rows)
            pltpu.sync_copy(rows, out_h.at[pl.ds(c * NP + s * RPS + t * C, C)])

    return k(dst3, zeros, ones)


def _sc_segsum_wide(feat_a, feat_b, src4, dst4, zeros):
    mesh = plsc.VectorSubcoreMesh(core_axis_name="c", subcore_axis_name="s")

    @functools.partial(
        pl.kernel,
        out_type=jax.ShapeDtypeStruct((NC * NP, D), _F32),
        mesh=mesh,
        scratch_types=[
            pltpu.VMEM((C,), jnp.int32),
            pltpu.VMEM((C,), jnp.int32),
            pltpu.VMEM((C,), jnp.int32),
            pltpu.VMEM((C,), jnp.int32),
            pltpu.VMEM((C, D), _F32),
            pltpu.VMEM((C, D), _F32),
            pltpu.VMEM_SHARED((NP, D), _F32),
            pltpu.SemaphoreType.DMA,
            pltpu.SemaphoreType.DMA,
        ],
    )
    def k(fa_h, fb_h, src_h, dst_h, zero_h, out_h,
          isa, ida, isb, idb, rows_a, rows_b, shared, sem_a, sem_b):
        c = lax.axis_index("c")
        s = lax.axis_index("s")
        pltpu.sync_copy(zero_h, rows_a)
        for t in range(NDR):
            pltpu.sync_copy(rows_a, shared.at[pl.ds(s * RPS + t * C, C)])
        plsc.subcore_barrier()

        def run(feat_h):
            for stage in range(2):
                pltpu.sync_copy(src_h.at[s, stage, 0], isa)
                pltpu.sync_copy(dst_h.at[s, stage, 0], ida)
                pltpu.async_copy(feat_h.at[isa], rows_a, sem_a)

                def body(i, carry):
                    j0 = 2 * i
                    pltpu.sync_copy(src_h.at[s, stage, j0 + 1], isb)
                    pltpu.sync_copy(dst_h.at[s, stage, j0 + 1], idb)
                    pltpu.async_copy(feat_h.at[isb], rows_b, sem_b)
                    pltpu.make_async_copy(feat_h.at[isa], rows_a, sem_a).wait()
                    pltpu.sync_copy(rows_a, shared.at[ida], add=True)

                    @pl.when(i < NCH // 2 - 1)
                    def _():
                        pltpu.sync_copy(src_h.at[s, stage, j0 + 2], isa)
                        pltpu.sync_copy(dst_h.at[s, stage, j0 + 2], ida)
                        pltpu.async_copy(feat_h.at[isa], rows_a, sem_a)

                    pltpu.make_async_copy(feat_h.at[isb], rows_b, sem_b).wait()
                    pltpu.sync_copy(rows_b, shared.at[idb], add=True)
                    return carry

                lax.fori_loop(0, NCH // 2, body, 0)

        @pl.when(c == 0)
        def _():
            run(fa_h)

        @pl.when(c == 1)
        def _():
            run(fb_h)

        plsc.subcore_barrier()
        for t in range(NDR):
            pltpu.sync_copy(shared.at[pl.ds(s * RPS + t * C, C)], rows_a)
            pltpu.sync_copy(rows_a, out_h.at[pl.ds(c * NP + s * RPS + t * C, C)])

    return k(feat_a, feat_b, src4, dst4, zeros)


def _dgT(a, b):
    return lax.dot_general(a, b, (((1,), (1,)), ((), ())),
                           preferred_element_type=_F32)


def _tc_layer(h, m0, m1, c0, c1, Wl, bl, Wr):
    din = h.shape[1]
    dout = Wl.shape[0]

    def body(h_r, m0_r, m1_r, c0_r, c1_r, wl_r, bl_r, wr_r, ho_r):
        cnt = jnp.maximum(c0_r[:, 0:1] + c1_r[:, 0:1], 1.0)
        agg = (m0_r[...] + m1_r[...]) / cnt
        hv = _dgT(h_r[...], wl_r[...]) + bl_r[...] + _dgT(agg, wr_r[...])
        ho_r[...] = jnp.maximum(hv, 0.0)

    in_specs = [
        pl.BlockSpec((BN, din), lambda i: (i, 0)),
        pl.BlockSpec((BN, 128), lambda i: (i, 0)),
        pl.BlockSpec((BN, 128), lambda i: (i, 0)),
        pl.BlockSpec((BN, 128), lambda i: (i, 0)),
        pl.BlockSpec((BN, 128), lambda i: (i, 0)),
        pl.BlockSpec((dout, din), lambda i: (0, 0)),
        pl.BlockSpec((1, dout), lambda i: (0, 0)),
        pl.BlockSpec((dout, 128), lambda i: (0, 0)),
    ]
    return pl.pallas_call(
        body,
        grid=(N // BN,),
        in_specs=in_specs,
        out_specs=pl.BlockSpec((BN, dout), lambda i: (i, 0)),
        out_shape=jax.ShapeDtypeStruct((N, dout), _F32),
    )(h, m0, m1, c0, c1, Wl, bl.reshape(1, -1), Wr)


def _tc_layer_split(h, ma, mb, c0, c1, Wl, bl, Wra, Wrb):
    din = h.shape[1]
    dout = Wl.shape[0]

    def body(h_r, ma_r, mb_r, c0_r, c1_r, wl_r, bl_r, wra_r, wrb_r, ho_r):
        cnt = jnp.maximum(c0_r[:, 0:1] + c1_r[:, 0:1], 1.0)
        agg = _dgT(ma_r[...] / cnt, wra_r[...]) + _dgT(mb_r[...] / cnt, wrb_r[...])
        hv = _dgT(h_r[...], wl_r[...]) + bl_r[...] + agg
        ho_r[...] = jnp.maximum(hv, 0.0)

    in_specs = [
        pl.BlockSpec((BN, din), lambda i: (i, 0)),
        pl.BlockSpec((BN, 128), lambda i: (i, 0)),
        pl.BlockSpec((BN, 128), lambda i: (i, 0)),
        pl.BlockSpec((BN, 128), lambda i: (i, 0)),
        pl.BlockSpec((BN, 128), lambda i: (i, 0)),
        pl.BlockSpec((dout, din), lambda i: (0, 0)),
        pl.BlockSpec((1, dout), lambda i: (0, 0)),
        pl.BlockSpec((dout, 128), lambda i: (0, 0)),
        pl.BlockSpec((dout, 128), lambda i: (0, 0)),
    ]
    return pl.pallas_call(
        body,
        grid=(N // BN,),
        in_specs=in_specs,
        out_specs=pl.BlockSpec((BN, dout), lambda i: (i, 0)),
        out_shape=jax.ShapeDtypeStruct((N, dout), _F32),
    )(h, ma, mb, c0, c1, Wl, bl.reshape(1, -1), Wra, Wrb)


def _tc_pool_mlp(h4, batch2, l1W, l1b, l2W, l2b, l3W, l3b, l4W, l4b):
    nblk = N // BN

    def body(h_r, b_r, w1, b1, w2, b2, w3, b3, w4, b4, out_r, acc, cacc):
        i = pl.program_id(0)

        @pl.when(i == 0)
        def _():
            acc[...] = jnp.zeros((G, D), _F32)
            cacc[...] = jnp.zeros((G, D), _F32)

        mask = (b_r[...] == lax.broadcasted_iota(jnp.int32, (BN, G), 1)).astype(_F32)
        acc[...] += lax.dot_general(mask, h_r[...], (((0,), (0,)), ((), ())),
                                    preferred_element_type=_F32,
                                    precision=lax.Precision.HIGHEST)
        cacc[...] += lax.dot_general(mask, jnp.ones((BN, D), _F32),
                                     (((0,), (0,)), ((), ())),
                                     preferred_element_type=_F32,
                                     precision=lax.Precision.HIGHEST)

        @pl.when(i == nblk - 1)
        def _():
            g = acc[...] / jnp.maximum(cacc[...], 1.0)
            g = jnp.maximum(_dgT(g, w1[...]) + b1[...], 0.0)
            g = jnp.maximum(_dgT(g, w2[...]) + b2[...], 0.0)
            g = jnp.maximum(_dgT(g, w3[...]) + b3[...], 0.0)
            o = lax.dot_general(w4[...], g, (((1,), (1,)), ((), ())),
                                preferred_element_type=_F32)
            out_r[...] = o + b4[...]

    in_specs = [
        pl.BlockSpec((BN, D), lambda i: (i, 0)),
        pl.BlockSpec((BN, 1), lambda i: (i, 0)),
        pl.BlockSpec((128, 128), lambda i: (0, 0)),
        pl.BlockSpec((1, 128), lambda i: (0, 0)),
        pl.BlockSpec((64, 128), lambda i: (0, 0)),
        pl.BlockSpec((1, 64), lambda i: (0, 0)),
        pl.BlockSpec((64, 64), lambda i: (0, 0)),
        pl.BlockSpec((1, 64), lambda i: (0, 0)),
        pl.BlockSpec((1, 64), lambda i: (0, 0)),
        pl.BlockSpec((1, 1), lambda i: (0, 0)),
    ]
    out = pl.pallas_call(
        body,
        grid=(nblk,),
        in_specs=in_specs,
        out_specs=pl.BlockSpec((1, G), lambda i: (0, 0)),
        out_shape=jax.ShapeDtypeStruct((1, G), _F32),
        scratch_shapes=[pltpu.VMEM((G, D), _F32), pltpu.VMEM((G, D), _F32)],
    )(h4, batch2,
      l1W, l1b.reshape(1, -1), l2W, l2b.reshape(1, -1),
      l3W, l3b.reshape(1, -1), l4W, l4b.reshape(1, -1))
    return out


def kernel(x, edge_index, batch,
           conv1_Wl, conv1_bl, conv1_Wr,
           conv2_Wl, conv2_bl, conv2_Wr,
           conv3_Wl, conv3_bl, conv3_Wr,
           conv4_Wl, conv4_bl, conv4_Wr,
           lin1_W, lin1_b, lin2_W, lin2_b,
           lin3_W, lin3_b, lin4_W, lin4_b):
    pad = EPAD - E
    src3 = jnp.concatenate(
        [edge_index[0], jnp.zeros((pad,), jnp.int32)]).reshape(NW, NCH, C)
    dst3 = jnp.concatenate(
        [edge_index[1], jnp.full((pad,), N, jnp.int32)]).reshape(NW, NCH, C)
    zeros128 = jnp.zeros((C, D), _F32)
    ones128 = jnp.ones((C, D), _F32)

    cnt = _sc_segcnt(dst3, zeros128, ones128)
    c0 = cnt[:N]
    c1 = cnt[NP:NP + N]
    zeros128 = zeros128 + cnt[0:1, 0:1] * 0.0

    def seg(feat):
        m = _sc_segsum(feat, src3, dst3, zeros128)
        return m[:N], m[NP:NP + N]

    src4 = src3.reshape(NS, 2, NCH, C)
    dst4 = dst3.reshape(NS, 2, NCH, C)

    m0, m1 = seg(x)
    h1 = _tc_layer(x, m0, m1, c0, c1, conv1_Wl, conv1_bl, conv1_Wr)
    mw = _sc_segsum_wide(h1[:, :128], h1[:, 128:], src4, dst4, zeros128)
    h2 = _tc_layer_split(h1, mw[:N], mw[NP:NP + N], c0, c1, conv2_Wl,
                         conv2_bl, conv2_Wr[:, :128], conv2_Wr[:, 128:])
    m0, m1 = seg(h2)
    h3 = _tc_layer(h2, m0, m1, c0, c1, conv3_Wl, conv3_bl, conv3_Wr)
    m0, m1 = seg(h3)
    h4 = _tc_layer(h3, m0, m1, c0, c1, conv4_Wl, conv4_bl, conv4_Wr)

    out = _tc_pool_mlp(h4, batch.reshape(N, 1),
                       lin1_W, lin1_b, lin2_W, lin2_b,
                       lin3_W, lin3_b, lin4_W, lin4_b)
    return out.reshape(G)
